# baseline (speedup 1.0000x reference)
import functools

import jax
import jax.numpy as jnp
from jax import lax
from jax.experimental import pallas as pl
from jax.experimental.pallas import tpu as pltpu
from jax.experimental.pallas import tpu_sc as plsc

_VOCAB = 1000
_DIM = 64
_BATCH = 16384
_HIST = 200
_NC, _NS = 2, 16
_NW = _NC * _NS
_BPW = _BATCH // _NW
_NBI = 8
_NB = 2
_NSUP = _BPW // _NBI
_NQ = _NBI // _NB


def _emb_body(x_hbm, w_hbm, out_hbm, table_sh, rows0, rows1,
              idx0, idx1, sem_g, sem_i0, sem_i1, sem_s0, sem_s1):
    cid = lax.axis_index("c")
    sid = lax.axis_index("s")
    wid = sid * _NC + cid
    base = wid * _BPW

    @pl.when(sid == 0)
    def _stage():
        pltpu.sync_copy(w_hbm, table_sh)
    plsc.subcore_barrier()

    rows_b = (rows0, rows1)
    idx_b = (idx0, idx1)
    sem_i = (sem_i0, sem_i1)
    sem_s = (sem_s0, sem_s1)

    def idx_sl(s):
        return x_hbm.at[pl.ds((base + s * _NBI) * _HIST, _NBI * _HIST)]

    def out_sl(s, q):
        return out_hbm.at[pl.ds(base + s * _NBI + q * _NB, _NB)]

    def sub_chunk(s, q, ib, skip_store_wait=False):
        b = q % 2
        if not skip_store_wait:
            pltpu.make_async_copy(rows_b[b], out_sl(s, q), sem_s[b]).wait()
        cps = [pltpu.async_copy(
                   table_sh.at[idx_b[ib].at[pl.ds((q * _NB + i) * _HIST + h0,
                                                  hn)]],
                   rows_b[b].at[i, pl.ds(h0, hn)], sem_g)
               for i in range(_NB) for h0, hn in ((0, 104), (104, 96))]
        for cp in cps:
            cp.wait()
        pltpu.async_copy(rows_b[b], out_sl(s, q), sem_s[b])

    h0 = pltpu.async_copy(idx_sl(0), idx0, sem_i0)
    pltpu.async_copy(idx_sl(1), idx1, sem_i1)
    h0.wait()
    for q in range(_NQ):
        sub_chunk(0, q, 0, skip_store_wait=(q < 2))
    pltpu.async_copy(idx_sl(2), idx0, sem_i0)

    def steady(k, carry):
        for ib, s in ((1, 1 + 2 * k), (0, 2 + 2 * k)):
            pltpu.make_async_copy(idx_sl(s), idx_b[ib], sem_i[ib]).wait()
            for q in range(_NQ):
                sub_chunk(s, q, ib)
            pltpu.async_copy(idx_sl(s + 2), idx_b[ib], sem_i[ib])
        return carry

    lax.fori_loop(0, (_NSUP - 4) // 2, steady, 0)

    for s, ib, pref in ((_NSUP - 3, 1, True), (_NSUP - 2, 0, False),
                        (_NSUP - 1, 1, False)):
        pltpu.make_async_copy(idx_sl(s), idx_b[ib], sem_i[ib]).wait()
        for q in range(_NQ):
            sub_chunk(s, q, ib)
        if pref:
            pltpu.async_copy(idx_sl(_NSUP - 1), idx_b[ib], sem_i[ib])
    for b in range(2):
        pltpu.make_async_copy(rows_b[b], out_sl(_NSUP - 1, 2 + b),
                              sem_s[b]).wait()


_emb = functools.partial(
    pl.kernel,
    mesh=plsc.VectorSubcoreMesh(core_axis_name="c", subcore_axis_name="s"),
    out_type=jax.ShapeDtypeStruct((_BATCH, _HIST, _DIM), jnp.float32),
    scratch_types=[
        pltpu.MemorySpace.VMEM_SHARED((_VOCAB, _DIM), jnp.float32),
        pltpu.VMEM((_NB, _HIST, _DIM), jnp.float32),
        pltpu.VMEM((_NB, _HIST, _DIM), jnp.float32),
        pltpu.VMEM((_NBI * _HIST,), jnp.int32),
        pltpu.VMEM((_NBI * _HIST,), jnp.int32),
        pltpu.SemaphoreType.DMA,
        pltpu.SemaphoreType.DMA,
        pltpu.SemaphoreType.DMA,
        pltpu.SemaphoreType.DMA,
        pltpu.SemaphoreType.DMA,
    ],
)(_emb_body)


def kernel(x, weight):
    return _emb(x.reshape(-1), weight)


# speedup vs baseline: 1.0006x; 1.0006x over previous
import functools

import jax
import jax.numpy as jnp
from jax import lax
from jax.experimental import pallas as pl
from jax.experimental.pallas import tpu as pltpu
from jax.experimental.pallas import tpu_sc as plsc

_VOCAB = 1000
_DIM = 64
_BATCH = 16384
_HIST = 200
_NC, _NS = 2, 16
_NW = _NC * _NS
_BPW = _BATCH // _NW
_NBI = 8
_NB = 2
_NSUP = _BPW // _NBI
_NQ = _NBI // _NB


def _emb_body(x_hbm, w_hbm, out_hbm, table_sh, rows0, rows1,
              idx0, idx1, sem_g, sem_i0, sem_i1, sem_s0, sem_s1):
    cid = lax.axis_index("c")
    sid = lax.axis_index("s")
    wid = sid * _NC + cid
    base = wid * _BPW

    for k in range(_NS):
        r0 = k * 64
        rn = 64 if k < _NS - 1 else _VOCAB - r0

        @pl.when(sid == k)
        def _stage(r0=r0, rn=rn):
            pltpu.sync_copy(w_hbm.at[pl.ds(r0, rn)],
                            table_sh.at[pl.ds(r0, rn)])
    plsc.subcore_barrier()

    rows_b = (rows0, rows1)
    idx_b = (idx0, idx1)
    sem_i = (sem_i0, sem_i1)
    sem_s = (sem_s0, sem_s1)

    def idx_sl(s):
        return x_hbm.at[pl.ds((base + s * _NBI) * _HIST, _NBI * _HIST)]

    def out_sl(s, q):
        return out_hbm.at[pl.ds(base + s * _NBI + q * _NB, _NB)]

    def sub_chunk(s, q, ib, skip_store_wait=False):
        b = q % 2
        if not skip_store_wait:
            pltpu.make_async_copy(rows_b[b], out_sl(s, q), sem_s[b]).wait()
        cps = [pltpu.async_copy(
                   table_sh.at[idx_b[ib].at[pl.ds((q * _NB + i) * _HIST + h0,
                                                  hn)]],
                   rows_b[b].at[i, pl.ds(h0, hn)], sem_g)
               for i in range(_NB) for h0, hn in ((0, 104), (104, 96))]
        for cp in cps:
            cp.wait()
        pltpu.async_copy(rows_b[b], out_sl(s, q), sem_s[b])

    h0 = pltpu.async_copy(idx_sl(0), idx0, sem_i0)
    pltpu.async_copy(idx_sl(1), idx1, sem_i1)
    h0.wait()
    for q in range(_NQ):
        sub_chunk(0, q, 0, skip_store_wait=(q < 2))
    pltpu.async_copy(idx_sl(2), idx0, sem_i0)

    def steady(k, carry):
        for ib, s in ((1, 1 + 2 * k), (0, 2 + 2 * k)):
            pltpu.make_async_copy(idx_sl(s), idx_b[ib], sem_i[ib]).wait()
            for q in range(_NQ):
                sub_chunk(s, q, ib)
            pltpu.async_copy(idx_sl(s + 2), idx_b[ib], sem_i[ib])
        return carry

    lax.fori_loop(0, (_NSUP - 4) // 2, steady, 0)

    for s, ib, pref in ((_NSUP - 3, 1, True), (_NSUP - 2, 0, False),
                        (_NSUP - 1, 1, False)):
        pltpu.make_async_copy(idx_sl(s), idx_b[ib], sem_i[ib]).wait()
        for q in range(_NQ):
            sub_chunk(s, q, ib)
        if pref:
            pltpu.async_copy(idx_sl(_NSUP - 1), idx_b[ib], sem_i[ib])
    for b in range(2):
        pltpu.make_async_copy(rows_b[b], out_sl(_NSUP - 1, 2 + b),
                              sem_s[b]).wait()


_emb = functools.partial(
    pl.kernel,
    mesh=plsc.VectorSubcoreMesh(core_axis_name="c", subcore_axis_name="s"),
    out_type=jax.ShapeDtypeStruct((_BATCH, _HIST, _DIM), jnp.float32),
    scratch_types=[
        pltpu.MemorySpace.VMEM_SHARED((_VOCAB, _DIM), jnp.float32),
        pltpu.VMEM((_NB, _HIST, _DIM), jnp.float32),
        pltpu.VMEM((_NB, _HIST, _DIM), jnp.float32),
        pltpu.VMEM((_NBI * _HIST,), jnp.int32),
        pltpu.VMEM((_NBI * _HIST,), jnp.int32),
        pltpu.SemaphoreType.DMA,
        pltpu.SemaphoreType.DMA,
        pltpu.SemaphoreType.DMA,
        pltpu.SemaphoreType.DMA,
        pltpu.SemaphoreType.DMA,
    ],
)(_emb_body)


def kernel(x, weight):
    return _emb(x.reshape(-1), weight)


# speedup vs baseline: 1.2140x; 1.2133x over previous
import functools

import jax
import jax.numpy as jnp
from jax import lax
from jax.experimental import pallas as pl
from jax.experimental.pallas import tpu as pltpu
from jax.experimental.pallas import tpu_sc as plsc

_VOCAB = 1000
_DIM = 64
_PADW = 128
_TOTAL = 16384 * 200
_NC, _NS = 2, 16
_NW = _NC * _NS
_PER_W = _TOTAL // _NW
_IDXW = 100
_KSUB = 4
_CHUNK = _KSUB * _IDXW
_NCH = _PER_W // _CHUNK


def _emb_body(x_hbm, w_hbm, out_hbm, table_sh, rows0, rows1,
              idx0, idx1, sem_g, sem_i0, sem_i1, sem_s0, sem_s1):
    cid = lax.axis_index("c")
    sid = lax.axis_index("s")
    wid = sid * _NC + cid
    base = wid * (_PER_W // _IDXW)

    @pl.when(sid == 0)
    def _stage():
        pltpu.sync_copy(w_hbm, table_sh)
    plsc.subcore_barrier()

    rows_b = (rows0, rows1)
    idx_b = (idx0, idx1)
    sem_i = (sem_i0, sem_i1)
    sem_s = (sem_s0, sem_s1)

    def idx_sl(ci):
        return x_hbm.at[pl.ds(base + ci * _KSUB, _KSUB)]

    def out_sl(ci):
        return out_hbm.at[pl.ds((base + ci * _KSUB) * _IDXW, _CHUNK)]

    def do_gathers(b):
        cps = [pltpu.async_copy(table_sh.at[idx_b[b].at[j]],
                                rows_b[b].at[pl.ds(j * _IDXW, _IDXW)], sem_g)
               for j in range(_KSUB)]
        for cp in cps:
            cp.wait()

    h0 = pltpu.async_copy(idx_sl(0), idx0, sem_i0)
    h1 = pltpu.async_copy(idx_sl(1), idx1, sem_i1)
    for b, h in ((0, h0), (1, h1)):
        h.wait()
        do_gathers(b)
        pltpu.async_copy(rows_b[b], out_sl(b), sem_s[b])
        pltpu.async_copy(idx_sl(b + 2), idx_b[b], sem_i[b])

    def steady(k, carry):
        ci2 = 2 + 2 * k
        for b in range(2):
            ci = ci2 + b
            pltpu.make_async_copy(idx_sl(ci), idx_b[b], sem_i[b]).wait()
            pltpu.make_async_copy(rows_b[b], out_sl(ci), sem_s[b]).wait()
            do_gathers(b)
            pltpu.async_copy(rows_b[b], out_sl(ci), sem_s[b])
            pltpu.async_copy(idx_sl(ci + 2), idx_b[b], sem_i[b])
        return carry

    lax.fori_loop(0, (_NCH - 4) // 2, steady, 0)

    for b in range(2):
        ci = _NCH - 2 + b
        pltpu.make_async_copy(idx_sl(ci), idx_b[b], sem_i[b]).wait()
        pltpu.make_async_copy(rows_b[b], out_sl(ci), sem_s[b]).wait()
        do_gathers(b)
        pltpu.async_copy(rows_b[b], out_sl(ci), sem_s[b])
    for b in range(2):
        pltpu.make_async_copy(rows_b[b], out_sl(_NCH - 2 + b), sem_s[b]).wait()


_emb = functools.partial(
    pl.kernel,
    mesh=plsc.VectorSubcoreMesh(core_axis_name="c", subcore_axis_name="s"),
    compiler_params=pltpu.CompilerParams(use_tc_tiling_on_sc=False),
    out_type=jax.ShapeDtypeStruct((_TOTAL, _PADW), jnp.float32),
    scratch_types=[
        pltpu.MemorySpace.VMEM_SHARED((_VOCAB, _PADW), jnp.float32),
        pltpu.VMEM((_CHUNK, _PADW), jnp.float32),
        pltpu.VMEM((_CHUNK, _PADW), jnp.float32),
        pltpu.VMEM((_KSUB, _IDXW), jnp.int32),
        pltpu.VMEM((_KSUB, _IDXW), jnp.int32),
        pltpu.SemaphoreType.DMA,
        pltpu.SemaphoreType.DMA,
        pltpu.SemaphoreType.DMA,
        pltpu.SemaphoreType.DMA,
        pltpu.SemaphoreType.DMA,
    ],
)(_emb_body)


def kernel(x, weight):
    xf = x.reshape(_TOTAL // _IDXW, _IDXW)
    wp = jnp.pad(weight, ((0, 0), (0, _PADW - _DIM)))
    out = _emb(xf, wp)
    return out.reshape(x.shape[0], x.shape[1], _PADW)[:, :, :_DIM]
